# baseline (speedup 1.0000x reference)
import jax
import jax.numpy as jnp
from jax.experimental import pallas as pl
from jax.experimental.pallas import tpu as pltpu

_NCHUNKS = 8


def _copy_body(in_ref, out_ref, sems):
    n = in_ref.shape[0]
    rows = n // _NCHUNKS
    for c in range(_NCHUNKS):
        pltpu.make_async_copy(
            in_ref.at[pl.ds(c * rows, rows), :],
            out_ref.at[pl.ds(c * rows, rows), :],
            sems.at[c],
        ).start()
    for c in range(_NCHUNKS):
        pltpu.make_async_copy(
            in_ref.at[pl.ds(c * rows, rows), :],
            out_ref.at[pl.ds(c * rows, rows), :],
            sems.at[c],
        ).wait()


def kernel(c_embeddings):
    n, d = c_embeddings.shape
    return pl.pallas_call(
        _copy_body,
        out_shape=jax.ShapeDtypeStruct((n, d), c_embeddings.dtype),
        in_specs=[pl.BlockSpec(memory_space=pl.ANY)],
        out_specs=pl.BlockSpec(memory_space=pl.ANY),
        scratch_shapes=[pltpu.SemaphoreType.DMA((_NCHUNKS,))],
    )(c_embeddings)


# speedup vs baseline: 16.9358x; 16.9358x over previous
import jax
import jax.numpy as jnp
from jax.experimental import pallas as pl
from jax.experimental.pallas import tpu as pltpu


def _copy_body(in_ref, out_ref):
    out_ref[...] = in_ref[...]


def kernel(c_embeddings):
    n, d = c_embeddings.shape
    wide = 128
    rows = n * d // wide
    x = c_embeddings.reshape(rows, wide)
    block_rows = 5000
    assert rows % block_rows == 0
    out = pl.pallas_call(
        _copy_body,
        out_shape=jax.ShapeDtypeStruct((rows, wide), x.dtype),
        grid=(rows // block_rows,),
        in_specs=[pl.BlockSpec((block_rows, wide), lambda i: (i, 0))],
        out_specs=pl.BlockSpec((block_rows, wide), lambda i: (i, 0)),
    )(x)
    return out.reshape(n, d)


# speedup vs baseline: 389.9728x; 23.0266x over previous
import jax
import jax.numpy as jnp
from jax.experimental import pallas as pl
from jax.experimental.pallas import tpu as pltpu


def _copy_body(in_ref, out_ref):
    out_ref[...] = in_ref[...]


def kernel(c_embeddings):
    n, d = c_embeddings.shape
    xt = c_embeddings.T
    bc = 65536
    grid = (pl.cdiv(n, bc),)
    out = pl.pallas_call(
        _copy_body,
        out_shape=jax.ShapeDtypeStruct((d, n), xt.dtype),
        grid=grid,
        in_specs=[pl.BlockSpec((d, bc), lambda i: (0, i))],
        out_specs=pl.BlockSpec((d, bc), lambda i: (0, i)),
    )(xt)
    return out.T
